# baseline (speedup 1.0000x reference)
import functools

import jax
import jax.numpy as jnp
from jax import lax
from jax.experimental import pallas as pl
from jax.experimental.pallas import tpu as pltpu
from jax.experimental.pallas import tpu_sc as plsc

B, S, H, HEADS, DH, FF, E, K = 2, 2048, 1024, 16, 64, 4096, 8, 2
T = B * S
EPS = 1e-12

BLK = 256
NB = 40
NPAD = NB * BLK
FFB = 2048
NFF = FF // FFB

NW = 32


def _ln_rows(z, g, b):
    m = jnp.mean(z, axis=-1, keepdims=True)
    v = jnp.mean((z - m) ** 2, axis=-1, keepdims=True)
    return (z - m) / jnp.sqrt(v + EPS) * g + b


def _qkv_body(x_ref, w_ref, b_ref, o_ref):
    x = x_ref[...].astype(jnp.bfloat16)
    acc = jnp.dot(x, w_ref[...], preferred_element_type=jnp.float32)
    o_ref[...] = (acc + b_ref[...]).astype(jnp.bfloat16)


def _qkv(x2d, wqkv16, bqkv):
    return pl.pallas_call(
        _qkv_body,
        grid=(T // 256,),
        in_specs=[
            pl.BlockSpec((256, H), lambda i: (i, 0)),
            pl.BlockSpec((H, 3 * H), lambda i: (0, 0)),
            pl.BlockSpec((1, 3 * H), lambda i: (0, 0)),
        ],
        out_specs=pl.BlockSpec((256, 3 * H), lambda i: (i, 0)),
        out_shape=jax.ShapeDtypeStruct((T, 3 * H), jnp.bfloat16),
    )(x2d, wqkv16, bqkv)


def _attn_body(q_ref, k_ref, v_ref, o_ref):
    q = q_ref[0]
    k = k_ref[0]
    s = lax.dot_general(q, k, (((1,), (1,)), ((), ())),
                        preferred_element_type=jnp.float32) * 0.125
    m = jnp.max(s, axis=1, keepdims=True)
    p = jnp.exp(s - m)
    l = jnp.sum(p, axis=1, keepdims=True)
    a = (p / l).astype(jnp.bfloat16)
    ctx = jnp.dot(a, v_ref[0], preferred_element_type=jnp.float32)
    o_ref[0] = ctx.astype(jnp.bfloat16)


def _attention(qkvh):
    return pl.pallas_call(
        _attn_body,
        grid=(B, HEADS),
        in_specs=[
            pl.BlockSpec((1, S, DH), lambda b, h: (h, b, 0)),
            pl.BlockSpec((1, S, DH), lambda b, h: (h + HEADS, b, 0)),
            pl.BlockSpec((1, S, DH), lambda b, h: (h + 2 * HEADS, b, 0)),
        ],
        out_specs=pl.BlockSpec((1, S, DH), lambda b, h: (h, b, 0)),
        out_shape=jax.ShapeDtypeStruct((HEADS, T, DH), jnp.bfloat16),
    )(qkvh, qkvh, qkvh)


def _oproj_body(ctx_ref, x_ref, wo_ref, bo_ref, g_ref, b_ref, wg_ref,
                ao_ref, w1_ref, i1_ref, i2_ref):
    ctx = ctx_ref[...]
    z = jnp.dot(ctx, wo_ref[...], preferred_element_type=jnp.float32)
    z = z + bo_ref[...] + x_ref[...]
    a = _ln_rows(z, g_ref[...], b_ref[...])
    ao_ref[...] = a
    a16 = a.astype(jnp.bfloat16)
    logits = jnp.dot(a16, wg_ref[...], preferred_element_type=jnp.float32)
    iota = lax.broadcasted_iota(jnp.int32, logits.shape, 1)
    m1 = jnp.max(logits, axis=1)
    sel1 = logits == m1[:, None]
    i1 = jnp.min(jnp.where(sel1, iota, E), axis=1)
    l2 = jnp.where(iota == i1[:, None], -1e30, logits)
    m2 = jnp.max(l2, axis=1)
    sel2 = l2 == m2[:, None]
    i2 = jnp.min(jnp.where(sel2, iota, E), axis=1)
    w1_ref[...] = 1.0 / (1.0 + jnp.exp(m2 - m1))
    i1_ref[...] = i1
    i2_ref[...] = i2


def _oproj_route(ctx, x2d, wo16, bo, ln_g, ln_b, wg):
    return pl.pallas_call(
        _oproj_body,
        grid=(T // 256,),
        in_specs=[
            pl.BlockSpec((256, H), lambda i: (i, 0)),
            pl.BlockSpec((256, H), lambda i: (i, 0)),
            pl.BlockSpec((H, H), lambda i: (0, 0)),
            pl.BlockSpec((1, H), lambda i: (0, 0)),
            pl.BlockSpec((1, H), lambda i: (0, 0)),
            pl.BlockSpec((1, H), lambda i: (0, 0)),
            pl.BlockSpec((H, E), lambda i: (0, 0)),
        ],
        out_specs=[
            pl.BlockSpec((256, H), lambda i: (i, 0)),
            pl.BlockSpec((256,), lambda i: (i,)),
            pl.BlockSpec((256,), lambda i: (i,)),
            pl.BlockSpec((256,), lambda i: (i,)),
        ],
        out_shape=[
            jax.ShapeDtypeStruct((T, H), jnp.float32),
            jax.ShapeDtypeStruct((T,), jnp.float32),
            jax.ShapeDtypeStruct((T,), jnp.int32),
            jax.ShapeDtypeStruct((T,), jnp.int32),
        ],
    )(ctx, x2d, wo16, bo, ln_g, ln_b, wg)


def _sc_gather_rows(table, idx, n_out):
    W = table.shape[1]
    GCH = 32 if W >= 1024 else 64
    per_w = n_out // NW
    nch = per_w // GCH
    idx3 = idx.reshape(NW, nch, GCH)
    mesh = plsc.VectorSubcoreMesh(core_axis_name="c", subcore_axis_name="s")

    @functools.partial(
        pl.kernel,
        mesh=mesh,
        out_type=jax.ShapeDtypeStruct((n_out, W), table.dtype),
        scratch_types=[
            pltpu.VMEM((nch, GCH), jnp.int32),
            pltpu.VMEM((GCH, W), table.dtype),
            pltpu.VMEM((GCH, W), table.dtype),
            pltpu.VMEM((GCH, W), table.dtype),
            pltpu.SemaphoreType.DMA,
            pltpu.SemaphoreType.DMA,
            pltpu.SemaphoreType.DMA,
            pltpu.SemaphoreType.DMA,
            pltpu.SemaphoreType.DMA,
            pltpu.SemaphoreType.DMA,
        ],
    )
    def gather_kernel(table_hbm, idx_hbm, out_hbm, idx_v, buf0, buf1, buf2,
                      gsem0, gsem1, gsem2, wsem0, wsem1, wsem2):
        wid = lax.axis_index("s") * 2 + lax.axis_index("c")
        base = wid * per_w
        pltpu.sync_copy(idx_hbm.at[wid], idx_v)
        bufs = (buf0, buf1, buf2)
        gsems = (gsem0, gsem1, gsem2)
        wsems = (wsem0, wsem1, wsem2)
        gc = [None, None, None]
        wr = [None, None, None]
        for c in range(min(2, nch)):
            gc[c] = pltpu.async_copy(table_hbm.at[idx_v.at[c]], bufs[c],
                                     gsems[c])
        for c in range(nch):
            bj = c % 3
            gc[bj].wait()
            wr[bj] = pltpu.async_copy(
                bufs[bj], out_hbm.at[pl.ds(base + c * GCH, GCH)], wsems[bj])
            nxt = c + 2
            if nxt < nch:
                bn = nxt % 3
                if wr[bn] is not None:
                    wr[bn].wait()
                    wr[bn] = None
                gc[bn] = pltpu.async_copy(table_hbm.at[idx_v.at[nxt]],
                                          bufs[bn], gsems[bn])
        for bj in range(3):
            if wr[bj] is not None:
                wr[bj].wait()

    return gather_kernel(table, idx3)


TPW = T // NW
NV = T // 16


def _sc_route(i1, i2):
    mesh = plsc.VectorSubcoreMesh(core_axis_name="c", subcore_axis_name="s")

    @functools.partial(
        pl.kernel,
        mesh=mesh,
        out_type=[
            jax.ShapeDtypeStruct((NPAD + 16,), jnp.int32),
            jax.ShapeDtypeStruct((2 * T,), jnp.int32),
            jax.ShapeDtypeStruct((48,), jnp.int32),
        ],
        scratch_types=[
            pltpu.VMEM((T,), jnp.int32),
            pltpu.VMEM((T,), jnp.int32),
            pltpu.VMEM((2, TPW), jnp.int32),
            pltpu.VMEM((2, TPW), jnp.int32),
            pltpu.VMEM((2, TPW), jnp.int32),
            pltpu.VMEM((2, TPW), jnp.int32),
            pltpu.VMEM((48,), jnp.int32),
            pltpu.SemaphoreType.DMA,
        ],
    )
    def route_kernel(i1_hbm, i2_hbm, gidx_hbm, dest_hbm, be_hbm,
                     i1_v, i2_v, dst_v, tok_v, pidx_v, pzero_v, be_v, sem):
        wid = lax.axis_index("s") * 2 + lax.axis_index("c")
        pltpu.sync_copy(i1_hbm, i1_v)
        pltpu.sync_copy(i2_hbm, i2_v)
        iota = lax.iota(jnp.int32, 16)
        myv0 = wid * (TPW // 16)

        def scan_pass(ref):
            def body(v, carry):
                tot, pref = carry
                val = ref[pl.ds(v * 16, 16)]
                before = (v < myv0).astype(jnp.int32)
                new_tot, new_pref = [], []
                for e in range(E):
                    cnt = jnp.sum((val == e).astype(jnp.int32))
                    new_tot.append(tot[e] + cnt)
                    new_pref.append(pref[e] + cnt * before)
                return (tuple(new_tot), tuple(new_pref))
            zeros = tuple(jnp.int32(0) for _ in range(E))
            return lax.fori_loop(0, NV, body, (zeros, zeros))

        tot1, pref1 = scan_pass(i1_v)
        tot2, pref2 = scan_pass(i2_v)

        counts = [tot1[e] + tot2[e] for e in range(E)]
        padded = [((counts[e] + BLK - 1) // BLK) * BLK for e in range(E)]
        offs = []
        run = jnp.int32(0)
        for e in range(E):
            offs.append(run)
            run = run + padded[e]
        total_used = run

        def chunk_dest(ref, starts, ci):
            cur = list(starts)
            base_tok = wid * TPW
            for vi in range(TPW // 16):
                val = ref[pl.ds(base_tok + vi * 16, 16)]
                dst = jnp.zeros((16,), jnp.int32)
                for e in range(E):
                    m = val == e
                    mi = m.astype(jnp.int32)
                    pre = plsc.cumsum(mi)
                    dst = jnp.where(m, cur[e] + pre - 1, dst)
                    cur[e] = cur[e] + jnp.sum(mi)
                dst_v[ci, pl.ds(vi * 16, 16)] = dst
                tok_v[ci, pl.ds(vi * 16, 16)] = base_tok + vi * 16 + iota

        chunk_dest(i1_v, [offs[e] + pref1[e] for e in range(E)], 0)
        chunk_dest(i2_v, [offs[e] + tot1[e] + pref2[e] for e in range(E)], 1)

        pltpu.sync_copy(dst_v.at[0], dest_hbm.at[pl.ds(wid * TPW, TPW)])
        pltpu.sync_copy(dst_v.at[1], dest_hbm.at[pl.ds(T + wid * TPW, TPW)])
        pltpu.async_copy(tok_v.at[0], gidx_hbm.at[dst_v.at[0]], sem).wait()
        pltpu.async_copy(tok_v.at[1], gidx_hbm.at[dst_v.at[1]], sem).wait()

        @pl.when(wid < E)
        def _():
            pstart = jnp.int32(0)
            npad = jnp.int32(0)
            for e in range(E):
                sel = (wid == e).astype(jnp.int32)
                pstart = pstart + sel * (offs[e] + counts[e])
                npad = npad + sel * (padded[e] - counts[e])
            for vi in range(2 * TPW // 16):
                lane = vi * 16 + iota
                slot = jnp.where(lane < npad, pstart + lane, NPAD)
                ci, off = divmod(vi * 16, TPW)
                pidx_v[ci, pl.ds(off, 16)] = slot
                pzero_v[ci, pl.ds(off, 16)] = jnp.zeros((16,), jnp.int32)
            pltpu.async_copy(pzero_v.at[0], gidx_hbm.at[pidx_v.at[0]],
                             sem).wait()
            pltpu.async_copy(pzero_v.at[1], gidx_hbm.at[pidx_v.at[1]],
                             sem).wait()

        @pl.when((wid >= E) & (wid < E + 16))
        def _():
            base = total_used + (wid - E) * TPW
            for vi in range(TPW // 16):
                slot = base + vi * 16 + iota
                slot = jnp.where(slot < NPAD, slot, NPAD)
                pidx_v[0, pl.ds(vi * 16, 16)] = slot
                pzero_v[0, pl.ds(vi * 16, 16)] = jnp.zeros((16,), jnp.int32)
            pltpu.async_copy(pzero_v.at[0], gidx_hbm.at[pidx_v.at[0]],
                             sem).wait()

        @pl.when(wid == 31)
        def _():
            ends = []
            run2 = jnp.int32(0)
            for e in range(E):
                run2 = run2 + padded[e]
                ends.append(run2)
            for vi in range(3):
                pos = (vi * 16 + iota) * BLK
                acc = jnp.zeros((16,), jnp.int32)
                for e in range(E):
                    acc = acc + (pos >= ends[e]).astype(jnp.int32)
                be_v[pl.ds(vi * 16, 16)] = jnp.minimum(acc, E - 1)
            pltpu.sync_copy(be_v, be_hbm)

    return route_kernel(i1, i2)


def _moe_body(be_ref, x_ref, w1_ref, b1_ref, w2_ref, b2_ref, y_ref):
    j = pl.program_id(0)
    x = x_ref[...]
    h = jnp.dot(x, w1_ref[0], preferred_element_type=jnp.float32)
    h = h + b1_ref[0]
    h = 0.5 * h * (1.0 + lax.erf(h * 0.7071067811865476))
    yp = jnp.dot(h, w2_ref[0], preferred_element_type=jnp.float32)
    bias2 = b2_ref[0] * (j == 0).astype(jnp.float32)
    y_ref[0] = (yp + bias2).astype(jnp.bfloat16)


def _moe_ffn(x_sorted, w1, b1, w2, b2, block_expert):
    grid_spec = pltpu.PrefetchScalarGridSpec(
        num_scalar_prefetch=1,
        grid=(NFF, NB),
        in_specs=[
            pl.BlockSpec((BLK, H), lambda j, i, be: (i, 0)),
            pl.BlockSpec((1, H, FFB), lambda j, i, be: (be[i], 0, j)),
            pl.BlockSpec((1, 1, FFB), lambda j, i, be: (be[i], 0, j)),
            pl.BlockSpec((1, FFB, H), lambda j, i, be: (be[i], j, 0)),
            pl.BlockSpec((1, 1, H), lambda j, i, be: (be[i], 0, 0)),
        ],
        out_specs=pl.BlockSpec((1, BLK, H), lambda j, i, be: (j, i, 0)),
    )
    return pl.pallas_call(
        _moe_body,
        grid_spec=grid_spec,
        out_shape=jax.ShapeDtypeStruct((NFF, NPAD, H), jnp.bfloat16),
    )(block_expert, x_sorted, w1, b1.reshape(E, 1, FF),
      w2, b2.reshape(E, 1, H))


def _combine_body(y0a_ref, y0b_ref, y1a_ref, y1b_ref, w1_ref, ao_ref,
                  g_ref, b_ref, o_ref):
    w1 = w1_ref[...][:, None]
    y0 = y0a_ref[...].astype(jnp.float32) + y0b_ref[...].astype(jnp.float32)
    y1 = y1a_ref[...].astype(jnp.float32) + y1b_ref[...].astype(jnp.float32)
    moe = w1 * y0 + (1.0 - w1) * y1
    o_ref[...] = _ln_rows(moe + ao_ref[...], g_ref[...], b_ref[...])


def _combine(y0a, y0b, y1a, y1b, w1, attn_out, ln2_g, ln2_b):
    row = pl.BlockSpec((256, H), lambda i: (i, 0))
    return pl.pallas_call(
        _combine_body,
        grid=(T // 256,),
        in_specs=[
            row, row, row, row,
            pl.BlockSpec((256,), lambda i: (i,)),
            row,
            pl.BlockSpec((1, H), lambda i: (0, 0)),
            pl.BlockSpec((1, H), lambda i: (0, 0)),
        ],
        out_specs=row,
        out_shape=jax.ShapeDtypeStruct((T, H), jnp.float32),
    )(y0a, y0b, y1a, y1b, w1, attn_out, ln2_g, ln2_b)


def kernel(hidden_states, params):
    p = params
    x2d = hidden_states.reshape(T, H)

    wqkv16 = jnp.concatenate([p["Wq"], p["Wk"], p["Wv"]], axis=1).astype(
        jnp.bfloat16)
    bqkv = jnp.concatenate([p["bq"], p["bk"], p["bv"]])[None, :]
    wo16 = p["Wo"].astype(jnp.bfloat16)

    qkv = _qkv(x2d, wqkv16, bqkv)
    qkvh = qkv.reshape(T, 3 * HEADS, DH).transpose(1, 0, 2)
    ctxh = _attention(qkvh)
    ctx = ctxh.transpose(1, 0, 2).reshape(T, H)
    attn_out, w1, i1, i2 = _oproj_route(
        ctx, x2d, wo16, p["bo"][None, :], p["ln_attn_g"][None, :],
        p["ln_attn_b"][None, :], p["Wg"].astype(jnp.bfloat16))

    e_all = jnp.concatenate([i1, i2])
    onehot = (e_all[:, None] == jnp.arange(E)[None, :]).astype(jnp.int32)
    csum = jnp.cumsum(onehot, axis=0)
    counts = csum[-1]
    rank = jnp.take_along_axis(csum, e_all[:, None], axis=1)[:, 0] - 1
    padded = ((counts + BLK - 1) // BLK) * BLK
    offs = jnp.concatenate([jnp.zeros((1,), jnp.int32),
                            jnp.cumsum(padded).astype(jnp.int32)])
    dest = offs[e_all] + rank
    gidx = jnp.zeros((NPAD,), jnp.int32).at[dest].set(
        jnp.concatenate([jnp.arange(T, dtype=jnp.int32)] * 2))
    block_expert = jnp.clip(
        jnp.searchsorted(offs, jnp.arange(NB, dtype=jnp.int32) * BLK,
                         side="right").astype(jnp.int32) - 1, 0, E - 1)

    x_sorted = _sc_gather_rows(attn_out, gidx, NPAD)
    y2 = _moe_ffn(x_sorted, p["W1"], p["b1"], p["W2"], p["b2"], block_expert)

    def _gather_half(yh):
        yw = lax.bitcast_convert_type(yh.reshape(NPAD, H // 2, 2), jnp.int32)
        pw = _sc_gather_rows(yw, dest, 2 * T)
        return lax.bitcast_convert_type(pw, jnp.bfloat16).reshape(2 * T, H)

    pa = _gather_half(y2[0])
    pb = _gather_half(y2[1])

    out = _combine(pa[:T], pb[:T], pa[T:], pb[T:], w1, attn_out,
                   p["ln2_g"][None, :], p["ln2_b"][None, :])
    return out.reshape(B, S, H)


# speedup vs baseline: 1.3257x; 1.3257x over previous
import functools

import jax
import jax.numpy as jnp
from jax import lax
from jax.experimental import pallas as pl
from jax.experimental.pallas import tpu as pltpu
from jax.experimental.pallas import tpu_sc as plsc

B, S, H, HEADS, DH, FF, E, K = 2, 2048, 1024, 16, 64, 4096, 8, 2
T = B * S
EPS = 1e-12

BLK = 256
NB = 40
NPAD = NB * BLK
FFB = 2048
NFF = FF // FFB

NW = 32


def _ln_rows(z, g, b):
    m = jnp.mean(z, axis=-1, keepdims=True)
    v = jnp.mean((z - m) ** 2, axis=-1, keepdims=True)
    return (z - m) / jnp.sqrt(v + EPS) * g + b


def _qkv_body(x_ref, w_ref, b_ref, o_ref):
    x = x_ref[...].astype(jnp.bfloat16)
    acc = jnp.dot(x, w_ref[...], preferred_element_type=jnp.float32)
    o_ref[...] = (acc + b_ref[...]).astype(jnp.bfloat16)


def _qkv(x2d, wqkv16, bqkv):
    return pl.pallas_call(
        _qkv_body,
        grid=(T // 256,),
        in_specs=[
            pl.BlockSpec((256, H), lambda i: (i, 0)),
            pl.BlockSpec((H, 3 * H), lambda i: (0, 0)),
            pl.BlockSpec((1, 3 * H), lambda i: (0, 0)),
        ],
        out_specs=pl.BlockSpec((256, 3 * H), lambda i: (i, 0)),
        out_shape=jax.ShapeDtypeStruct((T, 3 * H), jnp.bfloat16),
    )(x2d, wqkv16, bqkv)


def _attn_body(q_ref, k_ref, v_ref, o_ref):
    q = q_ref[0]
    k = k_ref[0]
    s = lax.dot_general(q, k, (((1,), (1,)), ((), ())),
                        preferred_element_type=jnp.float32) * 0.125
    m = jnp.max(s, axis=1, keepdims=True)
    p = jnp.exp(s - m)
    l = jnp.sum(p, axis=1, keepdims=True)
    a = (p / l).astype(jnp.bfloat16)
    ctx = jnp.dot(a, v_ref[0], preferred_element_type=jnp.float32)
    o_ref[0] = ctx.astype(jnp.bfloat16)


def _attention(qkvh):
    return pl.pallas_call(
        _attn_body,
        grid=(B, HEADS),
        in_specs=[
            pl.BlockSpec((1, S, DH), lambda b, h: (h, b, 0)),
            pl.BlockSpec((1, S, DH), lambda b, h: (h + HEADS, b, 0)),
            pl.BlockSpec((1, S, DH), lambda b, h: (h + 2 * HEADS, b, 0)),
        ],
        out_specs=pl.BlockSpec((1, S, DH), lambda b, h: (h, b, 0)),
        out_shape=jax.ShapeDtypeStruct((HEADS, T, DH), jnp.bfloat16),
    )(qkvh, qkvh, qkvh)


def _oproj_body(ctx_ref, x_ref, wo_ref, bo_ref, g_ref, b_ref, wg_ref,
                ao_ref, w1_ref, i1_ref, i2_ref):
    ctx = ctx_ref[...]
    z = jnp.dot(ctx, wo_ref[...], preferred_element_type=jnp.float32)
    z = z + bo_ref[...] + x_ref[...]
    a = _ln_rows(z, g_ref[...], b_ref[...])
    ao_ref[...] = a
    a16 = a.astype(jnp.bfloat16)
    logits = jnp.dot(a16, wg_ref[...], preferred_element_type=jnp.float32)
    iota = lax.broadcasted_iota(jnp.int32, logits.shape, 1)
    m1 = jnp.max(logits, axis=1)
    sel1 = logits == m1[:, None]
    i1 = jnp.min(jnp.where(sel1, iota, E), axis=1)
    l2 = jnp.where(iota == i1[:, None], -1e30, logits)
    m2 = jnp.max(l2, axis=1)
    sel2 = l2 == m2[:, None]
    i2 = jnp.min(jnp.where(sel2, iota, E), axis=1)
    w1_ref[...] = 1.0 / (1.0 + jnp.exp(m2 - m1))
    i1_ref[...] = i1
    i2_ref[...] = i2


def _oproj_route(ctx, x2d, wo16, bo, ln_g, ln_b, wg):
    return pl.pallas_call(
        _oproj_body,
        grid=(T // 256,),
        in_specs=[
            pl.BlockSpec((256, H), lambda i: (i, 0)),
            pl.BlockSpec((256, H), lambda i: (i, 0)),
            pl.BlockSpec((H, H), lambda i: (0, 0)),
            pl.BlockSpec((1, H), lambda i: (0, 0)),
            pl.BlockSpec((1, H), lambda i: (0, 0)),
            pl.BlockSpec((1, H), lambda i: (0, 0)),
            pl.BlockSpec((H, E), lambda i: (0, 0)),
        ],
        out_specs=[
            pl.BlockSpec((256, H), lambda i: (i, 0)),
            pl.BlockSpec((256,), lambda i: (i,)),
            pl.BlockSpec((256,), lambda i: (i,)),
            pl.BlockSpec((256,), lambda i: (i,)),
        ],
        out_shape=[
            jax.ShapeDtypeStruct((T, H), jnp.float32),
            jax.ShapeDtypeStruct((T,), jnp.float32),
            jax.ShapeDtypeStruct((T,), jnp.int32),
            jax.ShapeDtypeStruct((T,), jnp.int32),
        ],
    )(ctx, x2d, wo16, bo, ln_g, ln_b, wg)


def _sc_gather_rows(table, idx, n_out):
    W = table.shape[1]
    GCH = 32 if W >= 1024 else 64
    per_w = n_out // NW
    nch = per_w // GCH
    idx3 = idx.reshape(NW, nch, GCH)
    mesh = plsc.VectorSubcoreMesh(core_axis_name="c", subcore_axis_name="s")

    @functools.partial(
        pl.kernel,
        mesh=mesh,
        out_type=jax.ShapeDtypeStruct((n_out, W), table.dtype),
        scratch_types=[
            pltpu.VMEM((nch, GCH), jnp.int32),
            pltpu.VMEM((GCH, W), table.dtype),
            pltpu.VMEM((GCH, W), table.dtype),
            pltpu.VMEM((GCH, W), table.dtype),
            pltpu.SemaphoreType.DMA,
            pltpu.SemaphoreType.DMA,
            pltpu.SemaphoreType.DMA,
            pltpu.SemaphoreType.DMA,
            pltpu.SemaphoreType.DMA,
            pltpu.SemaphoreType.DMA,
        ],
    )
    def gather_kernel(table_hbm, idx_hbm, out_hbm, idx_v, buf0, buf1, buf2,
                      gsem0, gsem1, gsem2, wsem0, wsem1, wsem2):
        wid = lax.axis_index("s") * 2 + lax.axis_index("c")
        base = wid * per_w
        pltpu.sync_copy(idx_hbm.at[wid], idx_v)
        bufs = (buf0, buf1, buf2)
        gsems = (gsem0, gsem1, gsem2)
        wsems = (wsem0, wsem1, wsem2)
        gc = [None, None, None]
        wr = [None, None, None]
        for c in range(min(2, nch)):
            gc[c] = pltpu.async_copy(table_hbm.at[idx_v.at[c]], bufs[c],
                                     gsems[c])
        for c in range(nch):
            bj = c % 3
            gc[bj].wait()
            wr[bj] = pltpu.async_copy(
                bufs[bj], out_hbm.at[pl.ds(base + c * GCH, GCH)], wsems[bj])
            nxt = c + 2
            if nxt < nch:
                bn = nxt % 3
                if wr[bn] is not None:
                    wr[bn].wait()
                    wr[bn] = None
                gc[bn] = pltpu.async_copy(table_hbm.at[idx_v.at[nxt]],
                                          bufs[bn], gsems[bn])
        for bj in range(3):
            if wr[bj] is not None:
                wr[bj].wait()

    return gather_kernel(table, idx3)


TPW = T // NW
NV = T // 16


def _sc_route(i1, i2):
    mesh = plsc.VectorSubcoreMesh(core_axis_name="c", subcore_axis_name="s")

    @functools.partial(
        pl.kernel,
        mesh=mesh,
        out_type=[
            jax.ShapeDtypeStruct((NPAD + 16,), jnp.int32),
            jax.ShapeDtypeStruct((2 * T,), jnp.int32),
            jax.ShapeDtypeStruct((48,), jnp.int32),
        ],
        scratch_types=[
            pltpu.VMEM((T,), jnp.int32),
            pltpu.VMEM((T,), jnp.int32),
            pltpu.VMEM((2, TPW), jnp.int32),
            pltpu.VMEM((2, TPW), jnp.int32),
            pltpu.VMEM((2, TPW), jnp.int32),
            pltpu.VMEM((2, TPW), jnp.int32),
            pltpu.VMEM((48,), jnp.int32),
            pltpu.SemaphoreType.DMA,
        ],
    )
    def route_kernel(i1_hbm, i2_hbm, gidx_hbm, dest_hbm, be_hbm,
                     i1_v, i2_v, dst_v, tok_v, pidx_v, pzero_v, be_v, sem):
        wid = lax.axis_index("s") * 2 + lax.axis_index("c")
        pltpu.sync_copy(i1_hbm, i1_v)
        pltpu.sync_copy(i2_hbm, i2_v)
        iota = lax.iota(jnp.int32, 16)
        myv0 = wid * (TPW // 16)

        def scan_pass(ref):
            def body(v, carry):
                tot, pref = carry
                val = ref[pl.ds(v * 16, 16)]
                before = (v < myv0).astype(jnp.int32)
                new_tot, new_pref = [], []
                for e in range(E):
                    cnt = jnp.sum((val == e).astype(jnp.int32))
                    new_tot.append(tot[e] + cnt)
                    new_pref.append(pref[e] + cnt * before)
                return (tuple(new_tot), tuple(new_pref))
            zeros = tuple(jnp.int32(0) for _ in range(E))
            return lax.fori_loop(0, NV, body, (zeros, zeros))

        tot1, pref1 = scan_pass(i1_v)
        tot2, pref2 = scan_pass(i2_v)

        counts = [tot1[e] + tot2[e] for e in range(E)]
        padded = [((counts[e] + BLK - 1) // BLK) * BLK for e in range(E)]
        offs = []
        run = jnp.int32(0)
        for e in range(E):
            offs.append(run)
            run = run + padded[e]
        total_used = run

        def chunk_dest(ref, starts, ci):
            cur = list(starts)
            base_tok = wid * TPW
            for vi in range(TPW // 16):
                val = ref[pl.ds(base_tok + vi * 16, 16)]
                dst = jnp.zeros((16,), jnp.int32)
                for e in range(E):
                    m = val == e
                    mi = m.astype(jnp.int32)
                    pre = plsc.cumsum(mi)
                    dst = jnp.where(m, cur[e] + pre - 1, dst)
                    cur[e] = cur[e] + jnp.sum(mi)
                dst_v[ci, pl.ds(vi * 16, 16)] = dst
                tok_v[ci, pl.ds(vi * 16, 16)] = base_tok + vi * 16 + iota

        chunk_dest(i1_v, [offs[e] + pref1[e] for e in range(E)], 0)
        chunk_dest(i2_v, [offs[e] + tot1[e] + pref2[e] for e in range(E)], 1)

        pltpu.sync_copy(dst_v.at[0], dest_hbm.at[pl.ds(wid * TPW, TPW)])
        pltpu.sync_copy(dst_v.at[1], dest_hbm.at[pl.ds(T + wid * TPW, TPW)])
        pltpu.async_copy(tok_v.at[0], gidx_hbm.at[dst_v.at[0]], sem).wait()
        pltpu.async_copy(tok_v.at[1], gidx_hbm.at[dst_v.at[1]], sem).wait()

        @pl.when(wid < E)
        def _():
            pstart = jnp.int32(0)
            npad = jnp.int32(0)
            for e in range(E):
                sel = (wid == e).astype(jnp.int32)
                pstart = pstart + sel * (offs[e] + counts[e])
                npad = npad + sel * (padded[e] - counts[e])
            for vi in range(2 * TPW // 16):
                lane = vi * 16 + iota
                slot = jnp.where(lane < npad, pstart + lane, NPAD)
                ci, off = divmod(vi * 16, TPW)
                pidx_v[ci, pl.ds(off, 16)] = slot
                pzero_v[ci, pl.ds(off, 16)] = jnp.zeros((16,), jnp.int32)
            pltpu.async_copy(pzero_v.at[0], gidx_hbm.at[pidx_v.at[0]],
                             sem).wait()
            pltpu.async_copy(pzero_v.at[1], gidx_hbm.at[pidx_v.at[1]],
                             sem).wait()

        @pl.when((wid >= E) & (wid < E + 16))
        def _():
            base = total_used + (wid - E) * TPW
            for vi in range(TPW // 16):
                slot = base + vi * 16 + iota
                slot = jnp.where(slot < NPAD, slot, NPAD)
                pidx_v[0, pl.ds(vi * 16, 16)] = slot
                pzero_v[0, pl.ds(vi * 16, 16)] = jnp.zeros((16,), jnp.int32)
            pltpu.async_copy(pzero_v.at[0], gidx_hbm.at[pidx_v.at[0]],
                             sem).wait()

        @pl.when(wid == 31)
        def _():
            ends = []
            run2 = jnp.int32(0)
            for e in range(E):
                run2 = run2 + padded[e]
                ends.append(run2)
            for vi in range(3):
                pos = (vi * 16 + iota) * BLK
                acc = jnp.zeros((16,), jnp.int32)
                for e in range(E):
                    acc = acc + (pos >= ends[e]).astype(jnp.int32)
                be_v[pl.ds(vi * 16, 16)] = jnp.minimum(acc, E - 1)
            pltpu.sync_copy(be_v, be_hbm)

    return route_kernel(i1, i2)


def _moe_body(be_ref, x_ref, w1_hbm, b1_ref, w2_hbm, b2_ref, y_ref,
              w1s, w2s, prev_e, sem1, sem2):
    i = pl.program_id(0)
    e = be_ref[i]

    @pl.when((i == 0) | (e != prev_e[0]))
    def _():
        cp1 = pltpu.make_async_copy(w1_hbm.at[e], w1s, sem1)
        cp2 = pltpu.make_async_copy(w2_hbm.at[e], w2s, sem2)
        cp1.start()
        cp2.start()
        cp1.wait()
        cp2.wait()
        prev_e[0] = e

    x = x_ref[...]
    h = jnp.dot(x, w1s[...], preferred_element_type=jnp.float32)
    h = h + b1_ref[0]
    h = 0.5 * h * (1.0 + lax.erf(h * 0.7071067811865476))
    yp = jnp.dot(h, w2s[...], preferred_element_type=jnp.float32)
    y_ref[...] = (yp + b2_ref[0]).astype(jnp.bfloat16)


def _moe_ffn(x_sorted, w1, b1, w2, b2, block_expert):
    grid_spec = pltpu.PrefetchScalarGridSpec(
        num_scalar_prefetch=1,
        grid=(NB,),
        in_specs=[
            pl.BlockSpec((BLK, H), lambda i, be: (i, 0)),
            pl.BlockSpec(memory_space=pl.ANY),
            pl.BlockSpec((1, 1, FF), lambda i, be: (be[i], 0, 0)),
            pl.BlockSpec(memory_space=pl.ANY),
            pl.BlockSpec((1, 1, H), lambda i, be: (be[i], 0, 0)),
        ],
        out_specs=pl.BlockSpec((BLK, H), lambda i, be: (i, 0)),
        scratch_shapes=[
            pltpu.VMEM((H, FF), jnp.float32),
            pltpu.VMEM((FF, H), jnp.float32),
            pltpu.SMEM((1,), jnp.int32),
            pltpu.SemaphoreType.DMA,
            pltpu.SemaphoreType.DMA,
        ],
    )
    return pl.pallas_call(
        _moe_body,
        grid_spec=grid_spec,
        out_shape=jax.ShapeDtypeStruct((NPAD, H), jnp.bfloat16),
    )(block_expert, x_sorted, w1, b1.reshape(E, 1, FF),
      w2, b2.reshape(E, 1, H))


def _combine_body(y0_ref, y1_ref, w1_ref, ao_ref, g_ref, b_ref, o_ref):
    w1 = w1_ref[...][:, None]
    moe = (w1 * y0_ref[...].astype(jnp.float32)
           + (1.0 - w1) * y1_ref[...].astype(jnp.float32))
    o_ref[...] = _ln_rows(moe + ao_ref[...], g_ref[...], b_ref[...])


def _combine(y0, y1, w1, attn_out, ln2_g, ln2_b):
    row = pl.BlockSpec((256, H), lambda i: (i, 0))
    return pl.pallas_call(
        _combine_body,
        grid=(T // 256,),
        in_specs=[
            row, row,
            pl.BlockSpec((256,), lambda i: (i,)),
            row,
            pl.BlockSpec((1, H), lambda i: (0, 0)),
            pl.BlockSpec((1, H), lambda i: (0, 0)),
        ],
        out_specs=row,
        out_shape=jax.ShapeDtypeStruct((T, H), jnp.float32),
    )(y0, y1, w1, attn_out, ln2_g, ln2_b)


def kernel(hidden_states, params):
    p = params
    x2d = hidden_states.reshape(T, H)

    wqkv16 = jnp.concatenate([p["Wq"], p["Wk"], p["Wv"]], axis=1).astype(
        jnp.bfloat16)
    bqkv = jnp.concatenate([p["bq"], p["bk"], p["bv"]])[None, :]
    wo16 = p["Wo"].astype(jnp.bfloat16)

    qkv = _qkv(x2d, wqkv16, bqkv)
    qkvh = qkv.reshape(T, 3 * HEADS, DH).transpose(1, 0, 2)
    ctxh = _attention(qkvh)
    ctx = ctxh.transpose(1, 0, 2).reshape(T, H)
    attn_out, w1, i1, i2 = _oproj_route(
        ctx, x2d, wo16, p["bo"][None, :], p["ln_attn_g"][None, :],
        p["ln_attn_b"][None, :], p["Wg"].astype(jnp.bfloat16))

    e_all = jnp.concatenate([i1, i2])
    onehot = (e_all[:, None] == jnp.arange(E)[None, :]).astype(jnp.int32)
    csum = jnp.cumsum(onehot, axis=0)
    counts = csum[-1]
    rank = jnp.take_along_axis(csum, e_all[:, None], axis=1)[:, 0] - 1
    padded = ((counts + BLK - 1) // BLK) * BLK
    offs = jnp.concatenate([jnp.zeros((1,), jnp.int32),
                            jnp.cumsum(padded).astype(jnp.int32)])
    dest = offs[e_all] + rank
    gidx = jnp.zeros((NPAD,), jnp.int32).at[dest].set(
        jnp.concatenate([jnp.arange(T, dtype=jnp.int32)] * 2))
    block_expert = jnp.clip(
        jnp.searchsorted(offs, jnp.arange(NB, dtype=jnp.int32) * BLK,
                         side="right").astype(jnp.int32) - 1, 0, E - 1)

    x_sorted = _sc_gather_rows(attn_out, gidx, NPAD)
    y = _moe_ffn(x_sorted, p["W1"], p["b1"], p["W2"], p["b2"], block_expert)

    yw = lax.bitcast_convert_type(y.reshape(NPAD, H // 2, 2), jnp.int32)
    pw = _sc_gather_rows(yw, dest, 2 * T)
    y_pairs = lax.bitcast_convert_type(pw, jnp.bfloat16).reshape(2 * T, H)

    out = _combine(y_pairs[:T], y_pairs[T:], w1, attn_out,
                   p["ln2_g"][None, :], p["ln2_b"][None, :])
    return out.reshape(B, S, H)


# speedup vs baseline: 1.3782x; 1.0396x over previous
import functools

import jax
import jax.numpy as jnp
from jax import lax
from jax.experimental import pallas as pl
from jax.experimental.pallas import tpu as pltpu
from jax.experimental.pallas import tpu_sc as plsc

B, S, H, HEADS, DH, FF, E, K = 2, 2048, 1024, 16, 64, 4096, 8, 2
T = B * S
EPS = 1e-12

BLK = 256
NB = 40
NPAD = NB * BLK
FFB = 2048
NFF = FF // FFB

NW = 32


def _ln_rows(z, g, b):
    m = jnp.mean(z, axis=-1, keepdims=True)
    v = jnp.mean((z - m) ** 2, axis=-1, keepdims=True)
    return (z - m) / jnp.sqrt(v + EPS) * g + b


def _qkv_body(x_ref, w_ref, b_ref, o_ref):
    x = x_ref[...].astype(jnp.bfloat16)
    acc = jnp.dot(x, w_ref[...], preferred_element_type=jnp.float32)
    o_ref[...] = (acc + b_ref[...]).astype(jnp.bfloat16)


def _qkv(x2d, wqkv16, bqkv):
    return pl.pallas_call(
        _qkv_body,
        grid=(T // 256,),
        in_specs=[
            pl.BlockSpec((256, H), lambda i: (i, 0)),
            pl.BlockSpec((H, 3 * H), lambda i: (0, 0)),
            pl.BlockSpec((1, 3 * H), lambda i: (0, 0)),
        ],
        out_specs=pl.BlockSpec((256, 3 * H), lambda i: (i, 0)),
        out_shape=jax.ShapeDtypeStruct((T, 3 * H), jnp.bfloat16),
    )(x2d, wqkv16, bqkv)


def _attn_body(q_ref, k_ref, v_ref, o_ref):
    for hh in range(2):
        q = q_ref[hh]
        k = k_ref[hh]
        s = lax.dot_general(q, k, (((1,), (1,)), ((), ())),
                            preferred_element_type=jnp.float32) * 0.125
        m = jnp.max(s, axis=1, keepdims=True)
        p = jnp.exp(s - m)
        l = jnp.sum(p, axis=1, keepdims=True)
        a = (p / l).astype(jnp.bfloat16)
        ctx = jnp.dot(a, v_ref[hh], preferred_element_type=jnp.float32)
        o_ref[hh] = ctx.astype(jnp.bfloat16)


def _attention(qkvh):
    return pl.pallas_call(
        _attn_body,
        grid=(B, HEADS // 2),
        in_specs=[
            pl.BlockSpec((2, S, DH), lambda b, h: (h, b, 0)),
            pl.BlockSpec((2, S, DH), lambda b, h: (h + HEADS // 2, b, 0)),
            pl.BlockSpec((2, S, DH), lambda b, h: (h + HEADS, b, 0)),
        ],
        out_specs=pl.BlockSpec((2, S, DH), lambda b, h: (h, b, 0)),
        out_shape=jax.ShapeDtypeStruct((HEADS, T, DH), jnp.bfloat16),
    )(qkvh, qkvh, qkvh)


def _oproj_body(ctx_ref, x_ref, wo_ref, bo_ref, g_ref, b_ref, wg_ref,
                ao_ref, w1_ref, i1_ref, i2_ref):
    ctx = ctx_ref[...]
    z = jnp.dot(ctx, wo_ref[...], preferred_element_type=jnp.float32)
    z = z + bo_ref[...] + x_ref[...]
    a = _ln_rows(z, g_ref[...], b_ref[...])
    ao_ref[...] = a
    a16 = a.astype(jnp.bfloat16)
    logits = jnp.dot(a16, wg_ref[...], preferred_element_type=jnp.float32)
    iota = lax.broadcasted_iota(jnp.int32, logits.shape, 1)
    m1 = jnp.max(logits, axis=1)
    sel1 = logits == m1[:, None]
    i1 = jnp.min(jnp.where(sel1, iota, E), axis=1)
    l2 = jnp.where(iota == i1[:, None], -1e30, logits)
    m2 = jnp.max(l2, axis=1)
    sel2 = l2 == m2[:, None]
    i2 = jnp.min(jnp.where(sel2, iota, E), axis=1)
    w1_ref[...] = 1.0 / (1.0 + jnp.exp(m2 - m1))
    i1_ref[...] = i1
    i2_ref[...] = i2


def _oproj_route(ctx, x2d, wo16, bo, ln_g, ln_b, wg):
    return pl.pallas_call(
        _oproj_body,
        grid=(T // 256,),
        in_specs=[
            pl.BlockSpec((256, H), lambda i: (i, 0)),
            pl.BlockSpec((256, H), lambda i: (i, 0)),
            pl.BlockSpec((H, H), lambda i: (0, 0)),
            pl.BlockSpec((1, H), lambda i: (0, 0)),
            pl.BlockSpec((1, H), lambda i: (0, 0)),
            pl.BlockSpec((1, H), lambda i: (0, 0)),
            pl.BlockSpec((H, E), lambda i: (0, 0)),
        ],
        out_specs=[
            pl.BlockSpec((256, H), lambda i: (i, 0)),
            pl.BlockSpec((256,), lambda i: (i,)),
            pl.BlockSpec((256,), lambda i: (i,)),
            pl.BlockSpec((256,), lambda i: (i,)),
        ],
        out_shape=[
            jax.ShapeDtypeStruct((T, H), jnp.float32),
            jax.ShapeDtypeStruct((T,), jnp.float32),
            jax.ShapeDtypeStruct((T,), jnp.int32),
            jax.ShapeDtypeStruct((T,), jnp.int32),
        ],
    )(ctx, x2d, wo16, bo, ln_g, ln_b, wg)


def _sc_gather_rows(table, idx, n_out):
    W = table.shape[1]
    GCH = 32 if W >= 1024 else 64
    per_w = n_out // NW
    nch = per_w // GCH
    idx3 = idx.reshape(NW, nch, GCH)
    mesh = plsc.VectorSubcoreMesh(core_axis_name="c", subcore_axis_name="s")

    @functools.partial(
        pl.kernel,
        mesh=mesh,
        out_type=jax.ShapeDtypeStruct((n_out, W), table.dtype),
        scratch_types=[
            pltpu.VMEM((nch, GCH), jnp.int32),
            pltpu.VMEM((GCH, W), table.dtype),
            pltpu.VMEM((GCH, W), table.dtype),
            pltpu.VMEM((GCH, W), table.dtype),
            pltpu.SemaphoreType.DMA,
            pltpu.SemaphoreType.DMA,
            pltpu.SemaphoreType.DMA,
            pltpu.SemaphoreType.DMA,
            pltpu.SemaphoreType.DMA,
            pltpu.SemaphoreType.DMA,
        ],
    )
    def gather_kernel(table_hbm, idx_hbm, out_hbm, idx_v, buf0, buf1, buf2,
                      gsem0, gsem1, gsem2, wsem0, wsem1, wsem2):
        wid = lax.axis_index("s") * 2 + lax.axis_index("c")
        base = wid * per_w
        pltpu.sync_copy(idx_hbm.at[wid], idx_v)
        bufs = (buf0, buf1, buf2)
        gsems = (gsem0, gsem1, gsem2)
        wsems = (wsem0, wsem1, wsem2)
        gc = [None, None, None]
        wr = [None, None, None]
        for c in range(min(2, nch)):
            gc[c] = pltpu.async_copy(table_hbm.at[idx_v.at[c]], bufs[c],
                                     gsems[c])
        for c in range(nch):
            bj = c % 3
            gc[bj].wait()
            wr[bj] = pltpu.async_copy(
                bufs[bj], out_hbm.at[pl.ds(base + c * GCH, GCH)], wsems[bj])
            nxt = c + 2
            if nxt < nch:
                bn = nxt % 3
                if wr[bn] is not None:
                    wr[bn].wait()
                    wr[bn] = None
                gc[bn] = pltpu.async_copy(table_hbm.at[idx_v.at[nxt]],
                                          bufs[bn], gsems[bn])
        for bj in range(3):
            if wr[bj] is not None:
                wr[bj].wait()

    return gather_kernel(table, idx3)


TPW = T // NW
NV = T // 16


def _sc_route(i1, i2):
    mesh = plsc.VectorSubcoreMesh(core_axis_name="c", subcore_axis_name="s")

    @functools.partial(
        pl.kernel,
        mesh=mesh,
        out_type=[
            jax.ShapeDtypeStruct((NPAD + 16,), jnp.int32),
            jax.ShapeDtypeStruct((2 * T,), jnp.int32),
            jax.ShapeDtypeStruct((48,), jnp.int32),
        ],
        scratch_types=[
            pltpu.VMEM((T,), jnp.int32),
            pltpu.VMEM((T,), jnp.int32),
            pltpu.VMEM((2, TPW), jnp.int32),
            pltpu.VMEM((2, TPW), jnp.int32),
            pltpu.VMEM((2, TPW), jnp.int32),
            pltpu.VMEM((2, TPW), jnp.int32),
            pltpu.VMEM((48,), jnp.int32),
            pltpu.SemaphoreType.DMA,
        ],
    )
    def route_kernel(i1_hbm, i2_hbm, gidx_hbm, dest_hbm, be_hbm,
                     i1_v, i2_v, dst_v, tok_v, pidx_v, pzero_v, be_v, sem):
        wid = lax.axis_index("s") * 2 + lax.axis_index("c")
        pltpu.sync_copy(i1_hbm, i1_v)
        pltpu.sync_copy(i2_hbm, i2_v)
        iota = lax.iota(jnp.int32, 16)
        myv0 = wid * (TPW // 16)

        def scan_pass(ref):
            def body(v, carry):
                tot, pref = carry
                val = ref[pl.ds(v * 16, 16)]
                before = (v < myv0).astype(jnp.int32)
                new_tot, new_pref = [], []
                for e in range(E):
                    cnt = jnp.sum((val == e).astype(jnp.int32))
                    new_tot.append(tot[e] + cnt)
                    new_pref.append(pref[e] + cnt * before)
                return (tuple(new_tot), tuple(new_pref))
            zeros = tuple(jnp.int32(0) for _ in range(E))
            return lax.fori_loop(0, NV, body, (zeros, zeros))

        tot1, pref1 = scan_pass(i1_v)
        tot2, pref2 = scan_pass(i2_v)

        counts = [tot1[e] + tot2[e] for e in range(E)]
        padded = [((counts[e] + BLK - 1) // BLK) * BLK for e in range(E)]
        offs = []
        run = jnp.int32(0)
        for e in range(E):
            offs.append(run)
            run = run + padded[e]
        total_used = run

        def chunk_dest(ref, starts, ci):
            cur = list(starts)
            base_tok = wid * TPW
            for vi in range(TPW // 16):
                val = ref[pl.ds(base_tok + vi * 16, 16)]
                dst = jnp.zeros((16,), jnp.int32)
                for e in range(E):
                    m = val == e
                    mi = m.astype(jnp.int32)
                    pre = plsc.cumsum(mi)
                    dst = jnp.where(m, cur[e] + pre - 1, dst)
                    cur[e] = cur[e] + jnp.sum(mi)
                dst_v[ci, pl.ds(vi * 16, 16)] = dst
                tok_v[ci, pl.ds(vi * 16, 16)] = base_tok + vi * 16 + iota

        chunk_dest(i1_v, [offs[e] + pref1[e] for e in range(E)], 0)
        chunk_dest(i2_v, [offs[e] + tot1[e] + pref2[e] for e in range(E)], 1)

        pltpu.sync_copy(dst_v.at[0], dest_hbm.at[pl.ds(wid * TPW, TPW)])
        pltpu.sync_copy(dst_v.at[1], dest_hbm.at[pl.ds(T + wid * TPW, TPW)])
        pltpu.async_copy(tok_v.at[0], gidx_hbm.at[dst_v.at[0]], sem).wait()
        pltpu.async_copy(tok_v.at[1], gidx_hbm.at[dst_v.at[1]], sem).wait()

        @pl.when(wid < E)
        def _():
            pstart = jnp.int32(0)
            npad = jnp.int32(0)
            for e in range(E):
                sel = (wid == e).astype(jnp.int32)
                pstart = pstart + sel * (offs[e] + counts[e])
                npad = npad + sel * (padded[e] - counts[e])
            for vi in range(2 * TPW // 16):
                lane = vi * 16 + iota
                slot = jnp.where(lane < npad, pstart + lane, NPAD)
                ci, off = divmod(vi * 16, TPW)
                pidx_v[ci, pl.ds(off, 16)] = slot
                pzero_v[ci, pl.ds(off, 16)] = jnp.zeros((16,), jnp.int32)
            pltpu.async_copy(pzero_v.at[0], gidx_hbm.at[pidx_v.at[0]],
                             sem).wait()
            pltpu.async_copy(pzero_v.at[1], gidx_hbm.at[pidx_v.at[1]],
                             sem).wait()

        @pl.when((wid >= E) & (wid < E + 16))
        def _():
            base = total_used + (wid - E) * TPW
            for vi in range(TPW // 16):
                slot = base + vi * 16 + iota
                slot = jnp.where(slot < NPAD, slot, NPAD)
                pidx_v[0, pl.ds(vi * 16, 16)] = slot
                pzero_v[0, pl.ds(vi * 16, 16)] = jnp.zeros((16,), jnp.int32)
            pltpu.async_copy(pzero_v.at[0], gidx_hbm.at[pidx_v.at[0]],
                             sem).wait()

        @pl.when(wid == 31)
        def _():
            ends = []
            run2 = jnp.int32(0)
            for e in range(E):
                run2 = run2 + padded[e]
                ends.append(run2)
            for vi in range(3):
                pos = (vi * 16 + iota) * BLK
                acc = jnp.zeros((16,), jnp.int32)
                for e in range(E):
                    acc = acc + (pos >= ends[e]).astype(jnp.int32)
                be_v[pl.ds(vi * 16, 16)] = jnp.minimum(acc, E - 1)
            pltpu.sync_copy(be_v, be_hbm)

    return route_kernel(i1, i2)


def _moe_body(be_ref, x_ref, w1_hbm, b1_ref, w2_hbm, b2_ref, y_ref,
              w1s, w2s, prev_e, sem1, sem2):
    i = pl.program_id(0)
    e = be_ref[i]

    @pl.when((i == 0) | (e != prev_e[0]))
    def _():
        cp1 = pltpu.make_async_copy(w1_hbm.at[e], w1s, sem1)
        cp2 = pltpu.make_async_copy(w2_hbm.at[e], w2s, sem2)
        cp1.start()
        cp2.start()
        cp1.wait()
        cp2.wait()
        prev_e[0] = e

    x = x_ref[...]
    h = jnp.dot(x, w1s[...], preferred_element_type=jnp.float32)
    h = h + b1_ref[0]
    h = 0.5 * h * (1.0 + lax.erf(h * 0.7071067811865476))
    yp = jnp.dot(h, w2s[...], preferred_element_type=jnp.float32)
    y_ref[...] = (yp + b2_ref[0]).astype(jnp.bfloat16)


def _moe_ffn(x_sorted, w1, b1, w2, b2, block_expert):
    grid_spec = pltpu.PrefetchScalarGridSpec(
        num_scalar_prefetch=1,
        grid=(NB,),
        in_specs=[
            pl.BlockSpec((BLK, H), lambda i, be: (i, 0)),
            pl.BlockSpec(memory_space=pl.ANY),
            pl.BlockSpec((1, 1, FF), lambda i, be: (be[i], 0, 0)),
            pl.BlockSpec(memory_space=pl.ANY),
            pl.BlockSpec((1, 1, H), lambda i, be: (be[i], 0, 0)),
        ],
        out_specs=pl.BlockSpec((BLK, H), lambda i, be: (i, 0)),
        scratch_shapes=[
            pltpu.VMEM((H, FF), jnp.float32),
            pltpu.VMEM((FF, H), jnp.float32),
            pltpu.SMEM((1,), jnp.int32),
            pltpu.SemaphoreType.DMA,
            pltpu.SemaphoreType.DMA,
        ],
    )
    return pl.pallas_call(
        _moe_body,
        grid_spec=grid_spec,
        out_shape=jax.ShapeDtypeStruct((NPAD, H), jnp.bfloat16),
    )(block_expert, x_sorted, w1, b1.reshape(E, 1, FF),
      w2, b2.reshape(E, 1, H))


def _combine_body(y0_ref, y1_ref, w1_ref, ao_ref, g_ref, b_ref, o_ref):
    w1 = w1_ref[...][:, None]
    moe = (w1 * y0_ref[...].astype(jnp.float32)
           + (1.0 - w1) * y1_ref[...].astype(jnp.float32))
    o_ref[...] = _ln_rows(moe + ao_ref[...], g_ref[...], b_ref[...])


def _combine(y0, y1, w1, attn_out, ln2_g, ln2_b):
    row = pl.BlockSpec((256, H), lambda i: (i, 0))
    return pl.pallas_call(
        _combine_body,
        grid=(T // 256,),
        in_specs=[
            row, row,
            pl.BlockSpec((256,), lambda i: (i,)),
            row,
            pl.BlockSpec((1, H), lambda i: (0, 0)),
            pl.BlockSpec((1, H), lambda i: (0, 0)),
        ],
        out_specs=row,
        out_shape=jax.ShapeDtypeStruct((T, H), jnp.float32),
    )(y0, y1, w1, attn_out, ln2_g, ln2_b)


def kernel(hidden_states, params):
    p = params
    x2d = hidden_states.reshape(T, H)

    wqkv16 = jnp.concatenate([p["Wq"], p["Wk"], p["Wv"]], axis=1).astype(
        jnp.bfloat16)
    bqkv = jnp.concatenate([p["bq"], p["bk"], p["bv"]])[None, :]
    wo16 = p["Wo"].astype(jnp.bfloat16)

    qkv = _qkv(x2d, wqkv16, bqkv)
    qkvh = qkv.reshape(T, 3 * HEADS, DH).transpose(1, 0, 2)
    ctxh = _attention(qkvh)
    ctx = ctxh.transpose(1, 0, 2).reshape(T, H)
    attn_out, w1, i1, i2 = _oproj_route(
        ctx, x2d, wo16, p["bo"][None, :], p["ln_attn_g"][None, :],
        p["ln_attn_b"][None, :], p["Wg"].astype(jnp.bfloat16))

    e_all = jnp.concatenate([i1, i2])
    onehot = (e_all[:, None] == jnp.arange(E)[None, :]).astype(jnp.int32)
    csum = jnp.cumsum(onehot, axis=0)
    counts = csum[-1]
    rank = jnp.take_along_axis(csum, e_all[:, None], axis=1)[:, 0] - 1
    padded = ((counts + BLK - 1) // BLK) * BLK
    offs = jnp.concatenate([jnp.zeros((1,), jnp.int32),
                            jnp.cumsum(padded).astype(jnp.int32)])
    dest = offs[e_all] + rank
    gidx = jnp.zeros((NPAD,), jnp.int32).at[dest].set(
        jnp.concatenate([jnp.arange(T, dtype=jnp.int32)] * 2))
    block_expert = jnp.clip(
        jnp.searchsorted(offs, jnp.arange(NB, dtype=jnp.int32) * BLK,
                         side="right").astype(jnp.int32) - 1, 0, E - 1)

    x_sorted = _sc_gather_rows(attn_out, gidx, NPAD)
    y = _moe_ffn(x_sorted, p["W1"], p["b1"], p["W2"], p["b2"], block_expert)

    yw = lax.bitcast_convert_type(y.reshape(NPAD, H // 2, 2), jnp.int32)
    pw = _sc_gather_rows(yw, dest, 2 * T)
    y_pairs = lax.bitcast_convert_type(pw, jnp.bfloat16).reshape(2 * T, H)

    out = _combine(y_pairs[:T], y_pairs[T:], w1, attn_out,
                   p["ln2_g"][None, :], p["ln2_b"][None, :])
    return out.reshape(B, S, H)
